# baseline (speedup 1.0000x reference)
import jax
import jax.numpy as jnp
from jax.experimental import pallas as pl


_SB = 512


def _add_body(in_ref, emb_ref, out_ref):
    out_ref[...] = in_ref[...] + emb_ref[...][None, :, :]


def kernel(inputs, embeddings):
    B, S, D = inputs.shape
    pos = embeddings[:S]
    sb = _SB if S % _SB == 0 else S
    grid = (S // sb, B)
    return pl.pallas_call(
        _add_body,
        grid=grid,
        in_specs=[
            pl.BlockSpec((1, sb, D), lambda s, b: (b, s, 0)),
            pl.BlockSpec((sb, D), lambda s, b: (s, 0)),
        ],
        out_specs=pl.BlockSpec((1, sb, D), lambda s, b: (b, s, 0)),
        out_shape=jax.ShapeDtypeStruct((B, S, D), inputs.dtype),
    )(inputs, pos)
